# baseline (speedup 1.0000x reference)
import functools

import jax
import jax.numpy as jnp
from jax import lax
from jax.experimental import pallas as pl
from jax.experimental.pallas import tpu as pltpu
from jax.experimental.pallas import tpu_sc as plsc

_B, _H, _W = 32, 512, 512
_NLBL = _B * _H * _W
_NPRED = 2 * _NLBL
_SCALE = 1.0 / float(_NLBL)

_NW = 32
_LBL_PER_W = _NLBL // _NW
_CH = 8192
_STEPS = _LBL_PER_W // _CH
_GROUPS = _CH // 16

_mesh = plsc.VectorSubcoreMesh(core_axis_name="c", subcore_axis_name="s")


@functools.partial(
    pl.kernel,
    mesh=_mesh,
    out_type=jax.ShapeDtypeStruct((_NW, 16), jnp.float32),
    scratch_types=[
        pltpu.VMEM((2 * _CH,), jnp.float32),
        pltpu.VMEM((_CH,), jnp.float32),
        pltpu.VMEM((_CH,), jnp.float32),
        pltpu.VMEM((16,), jnp.float32),
    ],
    compiler_params=pltpu.CompilerParams(needs_layout_passes=False),
)
def _sc_partial(pred_hbm, reg_hbm, aff_hbm, out_hbm, predv, rv, av, accv):
    wid = lax.axis_index("s") * 2 + lax.axis_index("c")
    lbase = wid * _LBL_PER_W
    lane = lax.iota(jnp.int32, 16)

    def step_body(s, acc):
        off = lbase + s * _CH
        pltpu.sync_copy(pred_hbm.at[pl.ds(2 * off, 2 * _CH)], predv)
        pltpu.sync_copy(reg_hbm.at[pl.ds(off, _CH)], rv)
        pltpu.sync_copy(aff_hbm.at[pl.ds(off, _CH)], av)

        def grp(g, acc2):
            idx = 32 * g + 2 * lane
            p0 = plsc.load_gather(predv, [idx])
            p1 = plsc.load_gather(predv, [idx + 1])
            rr = rv[pl.ds(g * 16, 16)]
            aa = av[pl.ds(g * 16, 16)]
            d0 = p0 - rr
            d1 = p1 - aa
            return acc2 + d0 * d0 + d1 * d1

        return lax.fori_loop(0, _GROUPS, grp, acc)

    acc = lax.fori_loop(0, _STEPS, step_body, jnp.zeros(16, jnp.float32))
    accv[...] = acc
    pltpu.sync_copy(accv, out_hbm.at[wid])


def kernel(predicts, region_label, affinity_label):
    pred_f = predicts.reshape(_NPRED)
    reg_f = region_label.reshape(_NLBL)
    aff_f = affinity_label.reshape(_NLBL)
    parts = _sc_partial(pred_f, reg_f, aff_f)
    return jnp.sum(parts) * jnp.float32(_SCALE)


# speedup vs baseline: 31.5041x; 31.5041x over previous
import functools

import jax
import jax.numpy as jnp
from jax import lax
from jax.experimental import pallas as pl
from jax.experimental.pallas import tpu as pltpu
from jax.experimental.pallas import tpu_sc as plsc

_B, _H, _W = 32, 512, 512
_ROWS = _B * _H
_NLBL = _B * _H * _W
_SCALE = 1.0 / float(_NLBL)

_NW = 32
_ROWS_PER_W = _ROWS // _NW
_CR = 16
_STEPS = _ROWS_PER_W // _CR
_GROUPS = _W // 16

_mesh = plsc.VectorSubcoreMesh(core_axis_name="c", subcore_axis_name="s")


@functools.partial(
    pl.kernel,
    mesh=_mesh,
    out_type=jax.ShapeDtypeStruct((_NW, 16), jnp.float32),
    scratch_types=[
        pltpu.VMEM((_CR, 2 * _W), jnp.float32),
        pltpu.VMEM((_CR, _W), jnp.float32),
        pltpu.VMEM((_CR, _W), jnp.float32),
        pltpu.VMEM((16,), jnp.float32),
    ],
    compiler_params=pltpu.CompilerParams(needs_layout_passes=False),
)
def _sc_partial(pred_hbm, reg_hbm, aff_hbm, out_hbm, predv, rv, av, accv):
    wid = lax.axis_index("s") * 2 + lax.axis_index("c")
    rbase = wid * _ROWS_PER_W
    lane = lax.iota(jnp.int32, 16)

    def step_body(s, acc):
        row0 = rbase + s * _CR
        pltpu.sync_copy(pred_hbm.at[pl.ds(row0, _CR), :], predv)
        pltpu.sync_copy(reg_hbm.at[pl.ds(row0, _CR), :], rv)
        pltpu.sync_copy(aff_hbm.at[pl.ds(row0, _CR), :], av)

        def row_body(r, acc_r):
            def grp(g, acc2):
                col = 32 * g + 2 * lane
                row_splat = jnp.full((16,), r, jnp.int32)
                p0 = plsc.load_gather(predv, [row_splat, col])
                p1 = plsc.load_gather(predv, [row_splat, col + 1])
                rr = rv[r, pl.ds(g * 16, 16)]
                aa = av[r, pl.ds(g * 16, 16)]
                d0 = p0 - rr
                d1 = p1 - aa
                return acc2 + d0 * d0 + d1 * d1

            return lax.fori_loop(0, _GROUPS, grp, acc_r)

        return lax.fori_loop(0, _CR, row_body, acc)

    acc = lax.fori_loop(0, _STEPS, step_body, jnp.zeros(16, jnp.float32))
    accv[...] = acc
    pltpu.sync_copy(accv, out_hbm.at[wid])


def kernel(predicts, region_label, affinity_label):
    pred2d = predicts.reshape(_ROWS, 2 * _W)
    reg2d = region_label.reshape(_ROWS, _W)
    aff2d = affinity_label.reshape(_ROWS, _W)
    parts = _sc_partial(pred2d, reg2d, aff2d)
    return jnp.sum(parts) * jnp.float32(_SCALE)
